# Initial kernel scaffold; baseline (speedup 1.0000x reference)
import jax
import jax.numpy as jnp
from jax.experimental import pallas as pl


def kernel(node_embed, fixed_context, first_node, last_node, step_count, W_context_placeholder, W_dense):
    raise NotImplementedError("write your pallas kernel here")


# speedup vs baseline: 1622.0149x; 1622.0149x over previous
import functools

import jax
import jax.numpy as jnp
from jax import lax
from jax.experimental import pallas as pl
from jax.experimental.pallas import tpu as pltpu
from jax.experimental.pallas import tpu_sc as plsc

UNITS = 128
B = 4096
N = 200

_INFO = plsc.get_sparse_core_info()
_NC = _INFO.num_cores
_NS = _INFO.num_subcores
_NW = _NC * _NS
_BPW = B // _NW


def _sc_gather(table, gidx_f, gidx_l):
    mesh = plsc.VectorSubcoreMesh(core_axis_name="c", subcore_axis_name="s")

    @functools.partial(
        pl.kernel,
        mesh=mesh,
        out_type=(
            jax.ShapeDtypeStruct((B, UNITS), jnp.float32),
            jax.ShapeDtypeStruct((B, UNITS), jnp.float32),
        ),
        scratch_types=[
            pltpu.VMEM((_BPW,), jnp.int32),
            pltpu.VMEM((_BPW,), jnp.int32),
            pltpu.VMEM((_BPW, UNITS), jnp.float32),
            pltpu.VMEM((_BPW, UNITS), jnp.float32),
            pltpu.SemaphoreType.DMA,
            pltpu.SemaphoreType.DMA,
        ],
    )
    def k(table_hbm, gf_hbm, gl_hbm, outf_hbm, outl_hbm,
          idxf_v, idxl_v, rowsf_v, rowsl_v, semf, seml):
        wid = lax.axis_index("s") * _NC + lax.axis_index("c")
        base = wid * _BPW
        pltpu.sync_copy(gf_hbm.at[pl.ds(base, _BPW)], idxf_v)
        pltpu.sync_copy(gl_hbm.at[pl.ds(base, _BPW)], idxl_v)
        cpf = pltpu.async_copy(table_hbm.at[idxf_v], rowsf_v, semf)
        cpl = pltpu.async_copy(table_hbm.at[idxl_v], rowsl_v, seml)
        cpf.wait()
        cpl.wait()
        pltpu.sync_copy(rowsf_v, outf_hbm.at[pl.ds(base, _BPW)])
        pltpu.sync_copy(rowsl_v, outl_hbm.at[pl.ds(base, _BPW)])

    return k(table, gidx_f, gidx_l)


_BM = 512


def _proj_body(flag_ref, f_ref, l_ref, fc_ref, ph1_ref, ph2_ref,
               w1_ref, w2_ref, o_ref):
    use_ph = flag_ref[0] == 1
    f = jnp.where(use_ph, jnp.broadcast_to(ph1_ref[...], (_BM, UNITS)),
                  f_ref[...])
    l = jnp.where(use_ph, jnp.broadcast_to(ph2_ref[...], (_BM, UNITS)),
                  l_ref[...])
    acc = jnp.dot(f, w1_ref[...], preferred_element_type=jnp.float32)
    acc += jnp.dot(l, w2_ref[...], preferred_element_type=jnp.float32)
    o_ref[...] = fc_ref[...] + acc


def _tc_project(flag, first_rows, last_rows, fixed, ph1, ph2, w1, w2):
    grid = (B // _BM,)
    row_spec = pl.BlockSpec((_BM, UNITS), lambda i: (i, 0))
    full_spec = pl.BlockSpec((UNITS, UNITS), lambda i: (0, 0))
    ph_spec = pl.BlockSpec((1, UNITS), lambda i: (0, 0))
    return pl.pallas_call(
        _proj_body,
        grid=grid,
        in_specs=[
            pl.BlockSpec(memory_space=pltpu.SMEM),
            row_spec, row_spec, row_spec,
            ph_spec, ph_spec,
            full_spec, full_spec,
        ],
        out_specs=row_spec,
        out_shape=jax.ShapeDtypeStruct((B, UNITS), jnp.float32),
    )(flag, first_rows, last_rows, fixed, ph1, ph2, w1, w2)


def kernel(node_embed, fixed_context, first_node, last_node, step_count,
           W_context_placeholder, W_dense):
    table = node_embed.reshape(B * N, UNITS)
    offs = jnp.arange(B, dtype=jnp.int32) * N
    gidx_f = offs + first_node[:, 0].astype(jnp.int32)
    gidx_l = offs + last_node[:, 0].astype(jnp.int32)

    first_rows, last_rows = _sc_gather(table, gidx_f, gidx_l)

    flag = (jnp.asarray(step_count, jnp.int32) == 1).astype(jnp.int32)[None]
    ph1 = W_context_placeholder[:UNITS].reshape(1, UNITS)
    ph2 = W_context_placeholder[UNITS:].reshape(1, UNITS)
    w1 = W_dense[:UNITS]
    w2 = W_dense[UNITS:]
    fixed = fixed_context.reshape(B, UNITS)

    out = _tc_project(flag, first_rows, last_rows, fixed, ph1, ph2, w1, w2)
    return out.reshape(B, 1, UNITS)
